# baseline (speedup 1.0000x reference)
import functools

import jax
import jax.numpy as jnp
from jax import lax
from jax.experimental import pallas as pl
from jax.experimental.pallas import tpu as pltpu
from jax.experimental.pallas import tpu_sc as plsc

DIM = 1024
HID = 2816
NE = 8
TOPK = 2
SEQ = 2048

BLK = 256
HT = 1408
NH = HID // HT
RP = TOPK * SEQ + NE * BLK
G_SHARED = SEQ // BLK
G_ROUTED = RP // BLK
G = G_SHARED + G_ROUTED
R = SEQ + RP


L = 16
NTILE = 16
TPT = SEQ // NTILE
PPT = TOPK * TPT
FPT = R // NTILE
NW = 32
BPW = R // NW
GCH = 64


def _route_body(ltT, eid_out, wts0_out, wts1_out, hist_out,
                ltbuf, wtb0, wtb1, ei0, ei1, hbuf, cnts, eid_smem, sem):
    cid = lax.axis_index("c")
    sid = lax.axis_index("s")

    @pl.when(cid == 0)
    def _():
        j = sid
        lane = lax.iota(jnp.int32, L)

        for e in range(NE):
            pltpu.sync_copy(ltT.at[e, pl.ds(j * TPT, TPT)], ltbuf.at[e])
        for c in range(TPT // L):
            l = [ltbuf[e, pl.ds(c * L, L)] for e in range(NE)]
            b1 = l[0]
            i1 = jnp.zeros((L,), jnp.int32)
            b2 = jnp.full((L,), -1e30, jnp.float32)
            i2 = jnp.zeros((L,), jnp.int32)
            for e in range(1, NE):
                gt1 = l[e] > b1
                gt2 = l[e] > b2
                b2n = jnp.where(gt1, b1, jnp.where(gt2, l[e], b2))
                i2n = jnp.where(gt1, i1, jnp.where(gt2, e, i2))
                b1 = jnp.where(gt1, l[e], b1)
                i1 = jnp.where(gt1, e, i1)
                b2, i2 = b2n, i2n
            z = jnp.full((L,), 0.0, jnp.float32)
            for e in range(NE):
                z = z + jnp.exp(l[e] - b1)
            p1 = 1.0 / z
            p2 = jnp.exp(b2 - b1) / z
            s = p1 + p2 + 1e-8
            ei0[pl.ds(c * L, L)] = i1
            ei1[pl.ds(c * L, L)] = i2
            wtb0[pl.ds(c * L, L)] = p1 / s
            wtb1[pl.ds(c * L, L)] = p2 / s
            for li in range(L):
                eid_smem[c * L + li] = i1[li]
                eid_smem[TPT + c * L + li] = i2[li]
        pltpu.sync_copy(wtb0, wts0_out.at[pl.ds(j * TPT, TPT)])
        pltpu.sync_copy(wtb1, wts1_out.at[pl.ds(j * TPT, TPT)])
        pltpu.sync_copy(ei0, eid_out.at[pl.ds(j * PPT, TPT)])
        pltpu.sync_copy(ei1, eid_out.at[pl.ds(j * PPT + TPT, TPT)])

        for e in range(L):
            cnts[e] = 0

        def _hist_step(p, _):
            e = eid_smem[p]
            cnts[e] = cnts[e] + 1
            return 0

        lax.fori_loop(0, PPT, _hist_step, 0)
        hv = jnp.zeros((L,), jnp.int32)
        for e in range(NE):
            hv = jnp.where(lane == e, cnts[e], hv)
        hbuf[...] = hv
        pltpu.sync_copy(hbuf, hist_out.at[j])


def _place_body(eid_in, hist_in, gidx, pos0_out, pos1_out, emap_out,
                valid_out,
                posb0, posb1, fillb, hall, tokb, eidv, emapb, validb,
                eid_smem, pos_smem, cnts, sem):
    cid = lax.axis_index("c")
    sid = lax.axis_index("s")

    @pl.when(cid == 0)
    def _():
        j = sid
        lane = lax.iota(jnp.int32, L)

        for c in range(FPT // L):
            base = j * FPT + c * L
            v = base + lane
            fillb[pl.ds(c * L, L)] = jnp.where(v < SEQ, v, 0)
        pltpu.sync_copy(fillb, gidx.at[pl.ds(j * FPT, FPT)])

        for c in range(TPT // L):
            tokb[pl.ds(c * L, L)] = TPT * j + c * L + lane
        pltpu.sync_copy(eid_in.at[pl.ds(j * PPT, PPT)], eidv)
        for c in range(PPT // L):
            v = eidv[pl.ds(c * L, L)]
            for li in range(L):
                eid_smem[c * L + li] = v[li]
        pltpu.sync_copy(hist_in, hall)

        cntv = jnp.zeros((L,), jnp.int32)
        prefv = jnp.zeros((L,), jnp.int32)
        for jj in range(NTILE):
            rv = hall[jj]
            cntv = cntv + rv
            prefv = prefv + rv * jnp.where(jj < j, 1, 0)
        blk = jnp.int32(0)
        bo = []
        nb = []
        cnt = [cntv[e] for e in range(NE)]
        for e in range(NE):
            nbe = (cnt[e] + (BLK - 1)) // BLK
            bo.append(blk)
            nb.append(nbe)
            cnts[e] = SEQ + blk * BLK + prefv[e]
            blk = blk + nbe
        used = blk
        last_e = jnp.int32(0)
        for e in range(NE):
            last_e = jnp.where(cnt[e] > 0, jnp.int32(e), last_e)

        def _pos_step(p, _):
            e = eid_smem[p]
            r = cnts[e]
            cnts[e] = r + 1
            pos_smem[p] = r
            return 0

        lax.fori_loop(0, PPT, _pos_step, 0)
        for c in range(TPT // L):
            v0 = jnp.zeros((L,), jnp.int32)
            v1 = jnp.zeros((L,), jnp.int32)
            for li in range(L):
                v0 = jnp.where(lane == li, pos_smem[c * L + li], v0)
                v1 = jnp.where(lane == li, pos_smem[TPT + c * L + li], v1)
            posb0[pl.ds(c * L, L)] = v0
            posb1[pl.ds(c * L, L)] = v1
        pltpu.sync_copy(posb0, pos0_out.at[pl.ds(j * TPT, TPT)])
        pltpu.sync_copy(posb1, pos1_out.at[pl.ds(j * TPT, TPT)])

        pltpu.async_copy(tokb, gidx.at[posb0], sem).wait()
        pltpu.async_copy(tokb, gidx.at[posb1], sem).wait()

        @pl.when(j == 0)
        def _():
            for half in range(2):
                gv = half * L + lane
                gr = gv - G_SHARED
                routed_e = jnp.zeros((L,), jnp.int32)
                for e in range(NE):
                    own = (gr >= bo[e]) & (gr < bo[e] + nb[e])
                    routed_e = jnp.where(own, jnp.int32(e + 1), routed_e)
                vr = gr < used
                em = jnp.where(gv < G_SHARED, 0,
                               jnp.where(vr, routed_e, last_e + 1))
                vv = jnp.where((gv < G_SHARED) | vr, 1, 0)
                emapb[pl.ds(half * L, L)] = em
                validb[pl.ds(half * L, L)] = vv
            pltpu.sync_copy(emapb, emap_out)
            pltpu.sync_copy(validb, valid_out)


def _gather_body(x_hbm, gidx_hbm, xg_hbm, idxv, rows, sem):
    w = lax.axis_index("s") * 2 + lax.axis_index("c")
    for c in range(BPW // GCH):
        b = w * BPW + c * GCH
        pltpu.sync_copy(gidx_hbm.at[pl.ds(b, GCH)], idxv)
        pltpu.async_copy(x_hbm.at[idxv], rows, sem).wait()
        pltpu.sync_copy(rows, xg_hbm.at[pl.ds(b, GCH)])


_SC_MESH = plsc.VectorSubcoreMesh(core_axis_name="c", subcore_axis_name="s")


def _sc_dispatch(ltT):
    route = pl.kernel(
        _route_body,
        out_type=(
            jax.ShapeDtypeStruct((TOPK * SEQ,), jnp.int32),
            jax.ShapeDtypeStruct((SEQ,), jnp.float32),
            jax.ShapeDtypeStruct((SEQ,), jnp.float32),
            jax.ShapeDtypeStruct((NTILE, L), jnp.int32),
        ),
        mesh=_SC_MESH,
        scratch_types=[
            pltpu.VMEM((NE, TPT), jnp.float32),
            pltpu.VMEM((TPT,), jnp.float32),
            pltpu.VMEM((TPT,), jnp.float32),
            pltpu.VMEM((TPT,), jnp.int32),
            pltpu.VMEM((TPT,), jnp.int32),
            pltpu.VMEM((L,), jnp.int32),
            pltpu.SMEM((L,), jnp.int32),
            pltpu.SMEM((PPT,), jnp.int32),
            pltpu.SemaphoreType.DMA,
        ],
    )
    eid, wts0, wts1, hist = route(ltT)
    place = pl.kernel(
        _place_body,
        out_type=(
            jax.ShapeDtypeStruct((R,), jnp.int32),
            jax.ShapeDtypeStruct((SEQ,), jnp.int32),
            jax.ShapeDtypeStruct((SEQ,), jnp.int32),
            jax.ShapeDtypeStruct((G,), jnp.int32),
            jax.ShapeDtypeStruct((G,), jnp.int32),
        ),
        mesh=_SC_MESH,
        scratch_types=[
            pltpu.VMEM((TPT,), jnp.int32),
            pltpu.VMEM((TPT,), jnp.int32),
            pltpu.VMEM((FPT,), jnp.int32),
            pltpu.VMEM((NTILE, L), jnp.int32),
            pltpu.VMEM((TPT,), jnp.int32),
            pltpu.VMEM((PPT,), jnp.int32),
            pltpu.VMEM((G,), jnp.int32),
            pltpu.VMEM((G,), jnp.int32),
            pltpu.SMEM((PPT,), jnp.int32),
            pltpu.SMEM((PPT,), jnp.int32),
            pltpu.SMEM((L,), jnp.int32),
            pltpu.SemaphoreType.DMA,
        ],
    )
    gidx, pos0, pos1, e_map, valid = place(eid, hist)
    return gidx, pos0, pos1, wts0, wts1, e_map, valid


def _sc_gather(x2d, gidx):
    fn = pl.kernel(
        _gather_body,
        out_type=jax.ShapeDtypeStruct((R, DIM), jnp.float32),
        mesh=_SC_MESH,
        scratch_types=[
            pltpu.VMEM((GCH,), jnp.int32),
            pltpu.VMEM((GCH, DIM), jnp.float32),
            pltpu.SemaphoreType.DMA,
        ],
    )
    return fn(x2d, gidx)


def _ffn_half0_kernel(e_map_ref, valid_ref, x_ref, wg_ref, wu_ref, wd_ref,
                      out_ref):
    g = pl.program_id(0)

    @pl.when(valid_ref[g] > 0)
    def _():
        xb = x_ref[...]
        h = jnp.dot(xb, wg_ref[0], preferred_element_type=jnp.float32)
        u = jnp.dot(xb, wu_ref[0], preferred_element_type=jnp.float32)
        a = (h * jax.nn.sigmoid(h)) * u
        out_ref[...] = jnp.dot(a, wd_ref[0], preferred_element_type=jnp.float32)


def _ffn_half1_kernel(e_map_ref, valid_ref, x_ref, wg_ref, wu_ref, wd_ref,
                      p1_ref, out_ref):
    g = pl.program_id(0)

    @pl.when(valid_ref[g] > 0)
    def _():
        xb = x_ref[...]
        h = jnp.dot(xb, wg_ref[0], preferred_element_type=jnp.float32)
        u = jnp.dot(xb, wu_ref[0], preferred_element_type=jnp.float32)
        a = (h * jax.nn.sigmoid(h)) * u
        out_ref[...] = p1_ref[...] + jnp.dot(
            a, wd_ref[0], preferred_element_type=jnp.float32)


def _grouped_ffn(xg, wg, wu, wd, e_map, valid):
    def specs(ht):
        return [
            pl.BlockSpec((BLK, DIM), lambda g, em, vm: (g, 0)),
            pl.BlockSpec((1, DIM, HT), lambda g, em, vm: (em[g], 0, ht)),
            pl.BlockSpec((1, DIM, HT), lambda g, em, vm: (em[g], 0, ht)),
            pl.BlockSpec((1, HT, DIM), lambda g, em, vm: (em[g], ht, 0)),
        ]

    cp = pltpu.CompilerParams(dimension_semantics=("arbitrary",))
    out_sds = jax.ShapeDtypeStruct((R, DIM), jnp.float32)
    row_spec = pl.BlockSpec((BLK, DIM), lambda g, em, vm: (g, 0))
    p1 = pl.pallas_call(
        _ffn_half0_kernel,
        grid_spec=pltpu.PrefetchScalarGridSpec(
            num_scalar_prefetch=2, grid=(G,), in_specs=specs(0),
            out_specs=row_spec),
        out_shape=out_sds,
        compiler_params=cp,
    )(e_map, valid, xg, wg, wu, wd)
    return pl.pallas_call(
        _ffn_half1_kernel,
        grid_spec=pltpu.PrefetchScalarGridSpec(
            num_scalar_prefetch=2, grid=(G,), in_specs=specs(1) + [row_spec],
            out_specs=row_spec),
        out_shape=out_sds,
        compiler_params=cp,
    )(e_map, valid, xg, wg, wu, wd, p1)


def kernel(x, loop_idx, shared_wg, shared_wu, shared_wd, expert_wg, expert_wu,
           expert_wd, loop_table, router_w):
    B, S, D = x.shape
    x2d = x.reshape(S, D)

    loop_emb = jax.lax.dynamic_index_in_dim(loop_table, loop_idx, 0,
                                            keepdims=False)
    bias = loop_emb @ router_w[D:]
    logits = x2d @ router_w[:D] + bias

    gidx, pos0, pos1, wts0, wts1, e_map, valid = _sc_dispatch(logits.T)
    xg = _sc_gather(x2d, gidx)

    wg_all = jnp.concatenate([shared_wg[None], expert_wg], axis=0)
    wu_all = jnp.concatenate([shared_wu[None], expert_wu], axis=0)
    wd_all = jnp.concatenate([shared_wd[None], expert_wd], axis=0)

    rows = _grouped_ffn(xg, wg_all, wu_all, wd_all, e_map, valid)

    out = (rows[:SEQ]
           + wts0[:, None] * rows[pos0]
           + wts1[:, None] * rows[pos1])
    return out.reshape(B, S, D)


# speedup vs baseline: 1.2083x; 1.2083x over previous
import functools

import jax
import jax.numpy as jnp
from jax import lax
from jax.experimental import pallas as pl
from jax.experimental.pallas import tpu as pltpu
from jax.experimental.pallas import tpu_sc as plsc

DIM = 1024
HID = 2816
NE = 8
TOPK = 2
SEQ = 2048

BLK = 256
HT = 1408
NH = HID // HT
RP = TOPK * SEQ + NE * BLK
G_SHARED = SEQ // BLK
G_ROUTED = RP // BLK
G = G_SHARED + G_ROUTED
R = SEQ + RP


L = 16
NTILE = 16
TPT = SEQ // NTILE
PPT = TOPK * TPT
FPT = R // NTILE
NW = 32
BPW = R // NW
GCH = 64


def _route_body(ltT, eid_out, wts0_out, wts1_out, hist_out,
                ltbuf, wtb0, wtb1, ei0, ei1, hbuf, cnts, eid_smem, sem):
    cid = lax.axis_index("c")
    sid = lax.axis_index("s")

    @pl.when(cid == 0)
    def _():
        j = sid
        lane = lax.iota(jnp.int32, L)

        for e in range(NE):
            pltpu.sync_copy(ltT.at[e, pl.ds(j * TPT, TPT)], ltbuf.at[e])
        for c in range(TPT // L):
            l = [ltbuf[e, pl.ds(c * L, L)] for e in range(NE)]
            b1 = l[0]
            i1 = jnp.zeros((L,), jnp.int32)
            b2 = jnp.full((L,), -1e30, jnp.float32)
            i2 = jnp.zeros((L,), jnp.int32)
            for e in range(1, NE):
                gt1 = l[e] > b1
                gt2 = l[e] > b2
                b2n = jnp.where(gt1, b1, jnp.where(gt2, l[e], b2))
                i2n = jnp.where(gt1, i1, jnp.where(gt2, e, i2))
                b1 = jnp.where(gt1, l[e], b1)
                i1 = jnp.where(gt1, e, i1)
                b2, i2 = b2n, i2n
            z = jnp.full((L,), 0.0, jnp.float32)
            for e in range(NE):
                z = z + jnp.exp(l[e] - b1)
            p1 = 1.0 / z
            p2 = jnp.exp(b2 - b1) / z
            s = p1 + p2 + 1e-8
            ei0[pl.ds(c * L, L)] = i1
            ei1[pl.ds(c * L, L)] = i2
            wtb0[pl.ds(c * L, L)] = p1 / s
            wtb1[pl.ds(c * L, L)] = p2 / s
            for li in range(L):
                eid_smem[c * L + li] = i1[li]
                eid_smem[TPT + c * L + li] = i2[li]
        pltpu.sync_copy(wtb0, wts0_out.at[pl.ds(j * TPT, TPT)])
        pltpu.sync_copy(wtb1, wts1_out.at[pl.ds(j * TPT, TPT)])
        pltpu.sync_copy(ei0, eid_out.at[pl.ds(j * PPT, TPT)])
        pltpu.sync_copy(ei1, eid_out.at[pl.ds(j * PPT + TPT, TPT)])

        for e in range(L):
            cnts[e] = 0

        def _hist_step(p, _):
            e = eid_smem[p]
            cnts[e] = cnts[e] + 1
            return 0

        lax.fori_loop(0, PPT, _hist_step, 0)
        hv = jnp.zeros((L,), jnp.int32)
        for e in range(NE):
            hv = jnp.where(lane == e, cnts[e], hv)
        hbuf[...] = hv
        pltpu.sync_copy(hbuf, hist_out.at[j])


def _place_body(eid_in, hist_in, gidx, pos0_out, pos1_out, emap_out,
                valid_out,
                posb0, posb1, fillb, hall, tokb, eidv, emapb, validb,
                eid_smem, pos_smem, cnts, sem):
    cid = lax.axis_index("c")
    sid = lax.axis_index("s")

    @pl.when(cid == 0)
    def _():
        j = sid
        lane = lax.iota(jnp.int32, L)

        for c in range(FPT // L):
            base = j * FPT + c * L
            v = base + lane
            fillb[pl.ds(c * L, L)] = jnp.where(v < SEQ, v, 0)
        pltpu.sync_copy(fillb, gidx.at[pl.ds(j * FPT, FPT)])

        for c in range(TPT // L):
            tokb[pl.ds(c * L, L)] = TPT * j + c * L + lane
        pltpu.sync_copy(eid_in.at[pl.ds(j * PPT, PPT)], eidv)
        for c in range(PPT // L):
            v = eidv[pl.ds(c * L, L)]
            for li in range(L):
                eid_smem[c * L + li] = v[li]
        pltpu.sync_copy(hist_in, hall)

        cntv = jnp.zeros((L,), jnp.int32)
        prefv = jnp.zeros((L,), jnp.int32)
        for jj in range(NTILE):
            rv = hall[jj]
            cntv = cntv + rv
            prefv = prefv + rv * jnp.where(jj < j, 1, 0)
        blk = jnp.int32(0)
        bo = []
        nb = []
        cnt = [cntv[e] for e in range(NE)]
        for e in range(NE):
            nbe = (cnt[e] + (BLK - 1)) // BLK
            bo.append(blk)
            nb.append(nbe)
            cnts[e] = SEQ + blk * BLK + prefv[e]
            blk = blk + nbe
        used = blk
        last_e = jnp.int32(0)
        for e in range(NE):
            last_e = jnp.where(cnt[e] > 0, jnp.int32(e), last_e)

        def _pos_step(p, _):
            e = eid_smem[p]
            r = cnts[e]
            cnts[e] = r + 1
            pos_smem[p] = r
            return 0

        lax.fori_loop(0, PPT, _pos_step, 0)
        for c in range(TPT // L):
            v0 = jnp.zeros((L,), jnp.int32)
            v1 = jnp.zeros((L,), jnp.int32)
            for li in range(L):
                v0 = jnp.where(lane == li, pos_smem[c * L + li], v0)
                v1 = jnp.where(lane == li, pos_smem[TPT + c * L + li], v1)
            posb0[pl.ds(c * L, L)] = v0
            posb1[pl.ds(c * L, L)] = v1
        pltpu.sync_copy(posb0, pos0_out.at[pl.ds(j * TPT, TPT)])
        pltpu.sync_copy(posb1, pos1_out.at[pl.ds(j * TPT, TPT)])

        pltpu.async_copy(tokb, gidx.at[posb0], sem).wait()
        pltpu.async_copy(tokb, gidx.at[posb1], sem).wait()

        @pl.when(j == 0)
        def _():
            for half in range(2):
                gv = half * L + lane
                gr = gv - G_SHARED
                routed_e = jnp.zeros((L,), jnp.int32)
                for e in range(NE):
                    own = (gr >= bo[e]) & (gr < bo[e] + nb[e])
                    routed_e = jnp.where(own, jnp.int32(e + 1), routed_e)
                vr = gr < used
                em = jnp.where(gv < G_SHARED, 0,
                               jnp.where(vr, routed_e, last_e + 1))
                vv = jnp.where((gv < G_SHARED) | vr, 1, 0)
                emapb[pl.ds(half * L, L)] = em
                validb[pl.ds(half * L, L)] = vv
            pltpu.sync_copy(emapb, emap_out)
            pltpu.sync_copy(validb, valid_out)


def _gather_body(x_hbm, gidx_hbm, xg_hbm, idxv, rows, sem):
    w = lax.axis_index("s") * 2 + lax.axis_index("c")
    for c in range(BPW // GCH):
        b = w * BPW + c * GCH
        pltpu.sync_copy(gidx_hbm.at[pl.ds(b, GCH)], idxv)
        pltpu.async_copy(x_hbm.at[idxv], rows, sem).wait()
        pltpu.sync_copy(rows, xg_hbm.at[pl.ds(b, GCH)])


_SC_MESH = plsc.VectorSubcoreMesh(core_axis_name="c", subcore_axis_name="s")


def _sc_dispatch(ltT):
    route = pl.kernel(
        _route_body,
        out_type=(
            jax.ShapeDtypeStruct((TOPK * SEQ,), jnp.int32),
            jax.ShapeDtypeStruct((SEQ,), jnp.float32),
            jax.ShapeDtypeStruct((SEQ,), jnp.float32),
            jax.ShapeDtypeStruct((NTILE, L), jnp.int32),
        ),
        mesh=_SC_MESH,
        scratch_types=[
            pltpu.VMEM((NE, TPT), jnp.float32),
            pltpu.VMEM((TPT,), jnp.float32),
            pltpu.VMEM((TPT,), jnp.float32),
            pltpu.VMEM((TPT,), jnp.int32),
            pltpu.VMEM((TPT,), jnp.int32),
            pltpu.VMEM((L,), jnp.int32),
            pltpu.SMEM((L,), jnp.int32),
            pltpu.SMEM((PPT,), jnp.int32),
            pltpu.SemaphoreType.DMA,
        ],
    )
    eid, wts0, wts1, hist = route(ltT)
    place = pl.kernel(
        _place_body,
        out_type=(
            jax.ShapeDtypeStruct((R,), jnp.int32),
            jax.ShapeDtypeStruct((SEQ,), jnp.int32),
            jax.ShapeDtypeStruct((SEQ,), jnp.int32),
            jax.ShapeDtypeStruct((G,), jnp.int32),
            jax.ShapeDtypeStruct((G,), jnp.int32),
        ),
        mesh=_SC_MESH,
        scratch_types=[
            pltpu.VMEM((TPT,), jnp.int32),
            pltpu.VMEM((TPT,), jnp.int32),
            pltpu.VMEM((FPT,), jnp.int32),
            pltpu.VMEM((NTILE, L), jnp.int32),
            pltpu.VMEM((TPT,), jnp.int32),
            pltpu.VMEM((PPT,), jnp.int32),
            pltpu.VMEM((G,), jnp.int32),
            pltpu.VMEM((G,), jnp.int32),
            pltpu.SMEM((PPT,), jnp.int32),
            pltpu.SMEM((PPT,), jnp.int32),
            pltpu.SMEM((L,), jnp.int32),
            pltpu.SemaphoreType.DMA,
        ],
    )
    gidx, pos0, pos1, e_map, valid = place(eid, hist)
    return gidx, pos0, pos1, wts0, wts1, e_map, valid


def _sc_gather(x2d, gidx):
    fn = pl.kernel(
        _gather_body,
        out_type=jax.ShapeDtypeStruct((R, DIM), jnp.float32),
        mesh=_SC_MESH,
        scratch_types=[
            pltpu.VMEM((GCH,), jnp.int32),
            pltpu.VMEM((GCH, DIM), jnp.float32),
            pltpu.SemaphoreType.DMA,
        ],
    )
    return fn(x2d, gidx)


def _ffn_half0_kernel(e_map_ref, valid_ref, x_ref, wg_ref, wu_ref, wd_ref,
                      out_ref):
    g = pl.program_id(0)

    @pl.when(valid_ref[g] > 0)
    def _():
        xb = x_ref[...]
        h = jnp.dot(xb, wg_ref[0], preferred_element_type=jnp.float32)
        u = jnp.dot(xb, wu_ref[0], preferred_element_type=jnp.float32)
        a = (h * jax.nn.sigmoid(h)) * u
        out_ref[...] = jnp.dot(a, wd_ref[0], preferred_element_type=jnp.float32)


def _ffn_half1_kernel(e_map_ref, valid_ref, x_ref, wg_ref, wu_ref, wd_ref,
                      p1_ref, out_ref):
    g = pl.program_id(0)

    @pl.when(valid_ref[g] > 0)
    def _():
        xb = x_ref[...]
        h = jnp.dot(xb, wg_ref[0], preferred_element_type=jnp.float32)
        u = jnp.dot(xb, wu_ref[0], preferred_element_type=jnp.float32)
        a = (h * jax.nn.sigmoid(h)) * u
        out_ref[...] = p1_ref[...] + jnp.dot(
            a, wd_ref[0], preferred_element_type=jnp.float32)


def _grouped_ffn(xg, wg, wu, wd, e_map, valid):
    def specs(ht):
        return [
            pl.BlockSpec((BLK, DIM), lambda g, em, vm: (g, 0)),
            pl.BlockSpec((1, DIM, HT), lambda g, em, vm: (em[g], 0, ht)),
            pl.BlockSpec((1, DIM, HT), lambda g, em, vm: (em[g], 0, ht)),
            pl.BlockSpec((1, HT, DIM), lambda g, em, vm: (em[g], ht, 0)),
        ]

    cp = pltpu.CompilerParams(dimension_semantics=("arbitrary",))
    out_sds = jax.ShapeDtypeStruct((R, DIM), jnp.float32)
    row_spec = pl.BlockSpec((BLK, DIM), lambda g, em, vm: (g, 0))
    p1 = pl.pallas_call(
        _ffn_half0_kernel,
        grid_spec=pltpu.PrefetchScalarGridSpec(
            num_scalar_prefetch=2, grid=(G,), in_specs=specs(0),
            out_specs=row_spec),
        out_shape=out_sds,
        compiler_params=cp,
    )(e_map, valid, xg, wg, wu, wd)
    return pl.pallas_call(
        _ffn_half1_kernel,
        grid_spec=pltpu.PrefetchScalarGridSpec(
            num_scalar_prefetch=2, grid=(G,), in_specs=specs(1) + [row_spec],
            out_specs=row_spec),
        out_shape=out_sds,
        compiler_params=cp,
    )(e_map, valid, xg, wg, wu, wd, p1)


def kernel(x, loop_idx, shared_wg, shared_wu, shared_wd, expert_wg, expert_wu,
           expert_wd, loop_table, router_w):
    B, S, D = x.shape
    x2d = x.reshape(S, D)

    loop_emb = jax.lax.dynamic_index_in_dim(loop_table, loop_idx, 0,
                                            keepdims=False)
    bias = loop_emb @ router_w[D:]
    logits = x2d @ router_w[:D] + bias

    gidx, pos0, pos1, wts0, wts1, e_map, valid = _sc_dispatch(logits.T)
    xg = x2d[gidx]

    wg_all = jnp.concatenate([shared_wg[None], expert_wg], axis=0)
    wu_all = jnp.concatenate([shared_wu[None], expert_wu], axis=0)
    wd_all = jnp.concatenate([shared_wd[None], expert_wd], axis=0)

    rows = _grouped_ffn(xg, wg_all, wu_all, wd_all, e_map, valid)

    out = (rows[:SEQ]
           + wts0[:, None] * rows[pos0]
           + wts1[:, None] * rows[pos1])
    return out.reshape(B, S, D)
